# Initial kernel scaffold; baseline (speedup 1.0000x reference)
import jax
import jax.numpy as jnp
from jax.experimental import pallas as pl


def kernel(hidden_states, Wq, Wk, Wv, Wo):
    raise NotImplementedError("write your pallas kernel here")


# speedup vs baseline: 1.0524x; 1.0524x over previous
import jax
import jax.numpy as jnp
from jax.experimental import pallas as pl

B, S, HID = 2, 2048, 2048
H, KVH, D = 16, 4, 128
THETA = 10000.0

BM = 256
BQ = 256
BK = 256
SCALE = 1.0 / (D ** 0.5)


def _rope(x, cos, sin):
    rot = jnp.concatenate([-x[:, D // 2:], x[:, : D // 2]], axis=1)
    return x * cos + rot * sin


def _q_proj_kernel(x_ref, wq_ref, cos_ref, sin_ref, q_ref):
    x = x_ref[0]
    cos = cos_ref[...]
    sin = sin_ref[...]
    y = jnp.dot(x, wq_ref[...])
    for h in range(H):
        q_ref[0, h, :, :] = _rope(y[:, h * D:(h + 1) * D], cos, sin)


def _kv_proj_kernel(x_ref, wkv_ref, cos_ref, sin_ref, k_ref, v_ref):
    x = x_ref[0]
    cos = cos_ref[...]
    sin = sin_ref[...]
    y = jnp.dot(x, wkv_ref[...])
    for h in range(KVH):
        k_ref[0, h, :, :] = _rope(y[:, h * D:(h + 1) * D], cos, sin)
        v_ref[0, h, :, :] = y[:, (KVH + h) * D:(KVH + h + 1) * D]


def _flash_kernel(q_ref, k_ref, v_ref, o_ref):
    qi = pl.program_id(2)
    q = q_ref[0, 0]
    rows = qi * BQ + jax.lax.broadcasted_iota(jnp.int32, (BQ, BK), 0)

    m0 = jnp.full((BQ, 1), -1e30, jnp.float32)
    l0 = jnp.zeros((BQ, 1), jnp.float32)
    acc0 = jnp.zeros((BQ, D), jnp.float32)

    def body(kb, carry):
        m, l, acc = carry
        ks = k_ref[0, 0, pl.ds(kb * BK, BK), :]
        vs = v_ref[0, 0, pl.ds(kb * BK, BK), :]
        s = jax.lax.dot_general(q, ks, (((1,), (1,)), ((), ()))) * SCALE
        cols = kb * BK + jax.lax.broadcasted_iota(jnp.int32, (BQ, BK), 1)
        s = jnp.where(cols <= rows, s, -1e30)
        m_new = jnp.maximum(m, s.max(axis=1, keepdims=True))
        alpha = jnp.exp(m - m_new)
        p = jnp.exp(s - m_new)
        l_new = l * alpha + p.sum(axis=1, keepdims=True)
        acc_new = acc * alpha + jnp.dot(p, vs)
        return m_new, l_new, acc_new

    m, l, acc = jax.lax.fori_loop(0, qi + 1, body, (m0, l0, acc0))
    o_ref[0, 0] = acc / l


def _out_proj_kernel(x_ref, wo_ref, o_ref):
    x = jnp.concatenate([x_ref[0, h] for h in range(H)], axis=1)
    o_ref[0] = jnp.dot(x, wo_ref[...])


def kernel(hidden_states, Wq, Wk, Wv, Wo):
    inv_freq = 1.0 / (THETA ** (jnp.arange(0, D, 2, dtype=jnp.float32) / D))
    t = jnp.arange(S, dtype=jnp.float32)
    freqs = jnp.outer(t, inv_freq)
    emb = jnp.concatenate([freqs, freqs], axis=-1)
    cos = jnp.cos(emb)
    sin = jnp.sin(emb)
    wkv = jnp.concatenate([Wk, Wv], axis=1)

    q = pl.pallas_call(
        _q_proj_kernel,
        grid=(B, S // BM),
        in_specs=[
            pl.BlockSpec((1, BM, HID), lambda b, m: (b, m, 0)),
            pl.BlockSpec((HID, H * D), lambda b, m: (0, 0)),
            pl.BlockSpec((BM, D), lambda b, m: (m, 0)),
            pl.BlockSpec((BM, D), lambda b, m: (m, 0)),
        ],
        out_specs=pl.BlockSpec((1, H, BM, D), lambda b, m: (b, 0, m, 0)),
        out_shape=jax.ShapeDtypeStruct((B, H, S, D), jnp.float32),
    )(hidden_states, Wq, cos, sin)

    k, v = pl.pallas_call(
        _kv_proj_kernel,
        grid=(B, S // BM),
        in_specs=[
            pl.BlockSpec((1, BM, HID), lambda b, m: (b, m, 0)),
            pl.BlockSpec((HID, 2 * KVH * D), lambda b, m: (0, 0)),
            pl.BlockSpec((BM, D), lambda b, m: (m, 0)),
            pl.BlockSpec((BM, D), lambda b, m: (m, 0)),
        ],
        out_specs=[
            pl.BlockSpec((1, KVH, BM, D), lambda b, m: (b, 0, m, 0)),
            pl.BlockSpec((1, KVH, BM, D), lambda b, m: (b, 0, m, 0)),
        ],
        out_shape=[
            jax.ShapeDtypeStruct((B, KVH, S, D), jnp.float32),
            jax.ShapeDtypeStruct((B, KVH, S, D), jnp.float32),
        ],
    )(hidden_states, wkv, cos, sin)

    o = pl.pallas_call(
        _flash_kernel,
        grid=(B, H, S // BQ),
        in_specs=[
            pl.BlockSpec((1, 1, BQ, D), lambda b, h, i: (b, h, i, 0)),
            pl.BlockSpec((1, 1, S, D), lambda b, h, i: (b, h // (H // KVH), 0, 0)),
            pl.BlockSpec((1, 1, S, D), lambda b, h, i: (b, h // (H // KVH), 0, 0)),
        ],
        out_specs=pl.BlockSpec((1, 1, BQ, D), lambda b, h, i: (b, h, i, 0)),
        out_shape=jax.ShapeDtypeStruct((B, H, S, D), jnp.float32),
    )(q, k, v)

    out = pl.pallas_call(
        _out_proj_kernel,
        grid=(B, S // BM),
        in_specs=[
            pl.BlockSpec((1, H, BM, D), lambda b, m: (b, 0, m, 0)),
            pl.BlockSpec((HID, HID), lambda b, m: (0, 0)),
        ],
        out_specs=pl.BlockSpec((1, BM, HID), lambda b, m: (b, m, 0)),
        out_shape=jax.ShapeDtypeStruct((B, S, HID), jnp.float32),
    )(o, Wo)

    return out
